# baseline (speedup 1.0000x reference)
import functools

import jax
import jax.numpy as jnp
from jax import lax
from jax.experimental import pallas as pl
from jax.experimental.pallas import tpu as pltpu
from jax.experimental.pallas import tpu_sc as plsc

NC = 2
NS = 16
NW = NC * NS
CHUNK = 128
SLAB = 16

_mesh = plsc.VectorSubcoreMesh(core_axis_name="c", subcore_axis_name="s")


def _round_up(a, b):
    return (a + b - 1) // b * b


def _make_deg_kernel(pn, ch, d):
    rpt = pn // NS

    @functools.partial(
        pl.kernel,
        out_type=jax.ShapeDtypeStruct((NC, pn, d), jnp.float32),
        mesh=_mesh,
        scratch_types=[
            pltpu.VMEM((SLAB, CHUNK), jnp.int32),
            pltpu.VMEM((SLAB, CHUNK), jnp.int32),
            pltpu.VMEM((CHUNK, d), jnp.float32),
            pltpu.VMEM((CHUNK, d), jnp.float32),
            pltpu.VMEM_SHARED((pn, d), jnp.float32),
            pltpu.SemaphoreType.DMA,
            pltpu.SemaphoreType.DMA,
        ],
    )
    def deg_kernel(src_hbm, dst_hbm, ones_l_hbm, ones_r_hbm, zeros_hbm,
                   deg_hbm, srcv, dstv, ones_l, ones_r, acc, sem0, sem1):
        c = lax.axis_index("c")
        s = lax.axis_index("s")
        wid = s * NC + c
        r0 = s * rpt
        pltpu.sync_copy(zeros_hbm.at[pl.ds(r0, rpt)], acc.at[pl.ds(r0, rpt)])
        pltpu.sync_copy(ones_l_hbm, ones_l)
        pltpu.sync_copy(ones_r_hbm, ones_r)
        plsc.subcore_barrier()

        def slab(si, carry):
            base = wid * ch + si * SLAB
            pltpu.sync_copy(src_hbm.at[pl.ds(base, SLAB)], srcv)
            pltpu.sync_copy(dst_hbm.at[pl.ds(base, SLAB)], dstv)

            pltpu.async_copy(ones_l, acc.at[srcv.at[0]], sem0, add=True)
            pltpu.async_copy(ones_r, acc.at[dstv.at[0]], sem1, add=True)

            def step(j, c2):
                @pl.when(j < SLAB - 1)
                def _():
                    pltpu.async_copy(ones_l, acc.at[srcv.at[j + 1]], sem0, add=True)
                    pltpu.async_copy(ones_r, acc.at[dstv.at[j + 1]], sem1, add=True)
                pltpu.make_async_copy(ones_l, acc.at[srcv.at[j]], sem0).wait()
                pltpu.make_async_copy(ones_r, acc.at[dstv.at[j]], sem1).wait()
                return c2

            lax.fori_loop(0, SLAB, step, 0)
            return carry

        lax.fori_loop(0, ch // SLAB, slab, 0)
        plsc.subcore_barrier()
        pltpu.sync_copy(acc.at[pl.ds(r0, rpt)], deg_hbm.at[c, pl.ds(r0, rpt)])

    return deg_kernel


def _make_agg_kernel(pn, ch, d):
    rpt = pn // NS

    @functools.partial(
        pl.kernel,
        out_type=jax.ShapeDtypeStruct((NC, pn, d), jnp.float32),
        mesh=_mesh,
        scratch_types=[
            pltpu.VMEM((SLAB, CHUNK), jnp.int32),
            pltpu.VMEM((SLAB, CHUNK), jnp.int32),
            pltpu.VMEM((CHUNK, d), jnp.float32),
            pltpu.VMEM((CHUNK, d), jnp.float32),
            pltpu.VMEM_SHARED((pn, d), jnp.float32),
            pltpu.SemaphoreType.DMA,
            pltpu.SemaphoreType.DMA,
            pltpu.SemaphoreType.DMA,
            pltpu.SemaphoreType.DMA,
        ],
    )
    def agg_kernel(table_hbm, src_hbm, dst_hbm, zeros_hbm, out_hbm,
                   srcv, dstv, rows0, rows1, acc, semg0, semg1, sems0, sems1):
        c = lax.axis_index("c")
        s = lax.axis_index("s")
        wid = s * NC + c
        r0 = s * rpt
        pltpu.sync_copy(zeros_hbm.at[pl.ds(r0, rpt)], acc.at[pl.ds(r0, rpt)])
        plsc.subcore_barrier()

        def slab(si, carry):
            base = wid * ch + si * SLAB
            pltpu.sync_copy(src_hbm.at[pl.ds(base, SLAB)], srcv)
            pltpu.sync_copy(dst_hbm.at[pl.ds(base, SLAB)], dstv)
            pltpu.async_copy(table_hbm.at[srcv.at[0]], rows0, semg0)

            def step(jj, c2):
                j0 = 2 * jj
                j1 = j0 + 1
                pltpu.async_copy(table_hbm.at[srcv.at[j1]], rows1, semg1)
                pltpu.make_async_copy(table_hbm.at[srcv.at[j0]], rows0, semg0).wait()
                pltpu.sync_copy(rows0, acc.at[dstv.at[j0]], add=True)
                nxt = jnp.minimum(j0 + 2, SLAB - 1)
                pltpu.async_copy(table_hbm.at[srcv.at[nxt]], rows0, semg0)
                pltpu.make_async_copy(table_hbm.at[srcv.at[j1]], rows1, semg1).wait()
                pltpu.sync_copy(rows1, acc.at[dstv.at[j1]], add=True)
                return c2

            lax.fori_loop(0, SLAB // 2, step, 0)
            pltpu.make_async_copy(table_hbm.at[srcv.at[0]], rows0, semg0).wait()
            return carry

        lax.fori_loop(0, ch // SLAB, slab, 0)
        plsc.subcore_barrier()
        pltpu.sync_copy(acc.at[pl.ds(r0, rpt)], out_hbm.at[c, pl.ds(r0, rpt)])

    return agg_kernel


def _hprime_body(x_ref, deg_ref, w1_ref, out_ref):
    d = deg_ref[0, :, 0:1] + deg_ref[1, :, 0:1]
    norm = lax.rsqrt(jnp.maximum(d, 1.0))
    out_ref[...] = jnp.dot(x_ref[...] * norm, w1_ref[...],
                           preferred_element_type=jnp.float32)


def _mid_body(agg_ref, deg_ref, b1_ref, ws_ref, wn_ref, b2_ref, s_ref, p_ref):
    a = agg_ref[0] + agg_ref[1]
    din = deg_ref[0, :, 0:1] + deg_ref[1, :, 0:1]
    norm_in = lax.rsqrt(jnp.maximum(din, 1.0))
    h1 = jnp.maximum(a * norm_in + b1_ref[...], 0.0)
    s_ref[...] = jnp.dot(h1, ws_ref[...],
                         preferred_element_type=jnp.float32) + b2_ref[...]
    p_ref[...] = jnp.dot(h1, wn_ref[...], preferred_element_type=jnp.float32)


def _final_body(s_ref, nb_ref, deg_ref, out_ref):
    din = deg_ref[0, :, 0:1] + deg_ref[1, :, 0:1]
    inv = 1.0 / jnp.maximum(din, 1.0)
    out_ref[...] = s_ref[...] + (nb_ref[0] + nb_ref[1]) * inv


def kernel(x, edge_index, W1, b1, W_self, W_neigh, b2):
    n, d = x.shape
    e = edge_index.shape[1]
    pn = _round_up(n, 512)
    ch = _round_up(-(-e // (NW * CHUNK)), SLAB)
    pad_e = NW * ch * CHUNK

    src = edge_index[0].astype(jnp.int32)
    dst = edge_index[1].astype(jnp.int32)
    n_sent = pn - n
    sent = n + (jnp.arange(pad_e - e, dtype=jnp.int32) % n_sent)
    src2d = jnp.concatenate([src, sent]).reshape(NW * ch, CHUNK)
    dst2d = jnp.concatenate([dst, sent]).reshape(NW * ch, CHUNK)
    x_pad = jnp.concatenate([x, jnp.zeros((pn - n, d), x.dtype)])

    zeros_d = jnp.zeros((pn, d), jnp.float32)
    hd = d // 2
    ones_l = jnp.concatenate([jnp.ones((CHUNK, hd), jnp.float32),
                              jnp.zeros((CHUNK, hd), jnp.float32)], axis=1)
    ones_r = ones_l[:, ::-1]

    agg_kern = _make_agg_kernel(pn, ch, d)
    deg = _make_deg_kernel(pn, ch, d)(src2d, dst2d, ones_l, ones_r, zeros_d)
    deg_o_col = deg[:, :, 0:8]
    deg_i_col = deg[:, :, 64:72]

    br = pn
    grid = 1
    row_spec = pl.BlockSpec((br, d), lambda i: (i, 0))
    two_spec = pl.BlockSpec((NC, br, d), lambda i: (0, i, 0))
    deg_spec = pl.BlockSpec((NC, br, 8), lambda i: (0, i, 0))
    w_spec = pl.BlockSpec((d, d), lambda i: (0, 0))
    b_spec = pl.BlockSpec((1, d), lambda i: (0, 0))
    fs = jax.ShapeDtypeStruct((pn, d), jnp.float32)

    hprime = pl.pallas_call(
        _hprime_body, grid=(grid,),
        in_specs=[row_spec, deg_spec, w_spec],
        out_specs=row_spec, out_shape=fs,
    )(x_pad, deg_o_col, W1)

    agg = agg_kern(hprime, src2d, dst2d, zeros_d)

    s_out, p_out = pl.pallas_call(
        _mid_body, grid=(grid,),
        in_specs=[two_spec, deg_spec, b_spec, w_spec, w_spec, b_spec],
        out_specs=(row_spec, row_spec), out_shape=(fs, fs),
    )(agg, deg_i_col, b1.reshape(1, d), W_self, W_neigh, b2.reshape(1, d))

    neigh = agg_kern(p_out, src2d, dst2d, zeros_d)

    h2 = pl.pallas_call(
        _final_body, grid=(grid,),
        in_specs=[row_spec, two_spec, deg_spec],
        out_specs=row_spec, out_shape=fs,
    )(s_out, neigh, deg_i_col)

    return h2[:n]


# speedup vs baseline: 1.0037x; 1.0037x over previous
import functools

import jax
import jax.numpy as jnp
from jax import lax
from jax.experimental import pallas as pl
from jax.experimental.pallas import tpu as pltpu
from jax.experimental.pallas import tpu_sc as plsc

NC = 2
NS = 16
NW = NC * NS
CHUNK = 128
SLAB = 16

_mesh = plsc.VectorSubcoreMesh(core_axis_name="c", subcore_axis_name="s")


def _round_up(a, b):
    return (a + b - 1) // b * b


def _make_deg_kernel(pn, ch, d):
    rpt = pn // NS

    @functools.partial(
        pl.kernel,
        out_type=jax.ShapeDtypeStruct((NC, pn, d), jnp.float32),
        mesh=_mesh,
        scratch_types=[
            pltpu.VMEM((SLAB, CHUNK), jnp.int32),
            pltpu.VMEM((SLAB, CHUNK), jnp.int32),
            pltpu.VMEM((CHUNK, d), jnp.float32),
            pltpu.VMEM((CHUNK, d), jnp.float32),
            pltpu.VMEM_SHARED((pn, d), jnp.float32),
            pltpu.SemaphoreType.DMA,
            pltpu.SemaphoreType.DMA,
        ],
    )
    def deg_kernel(src_hbm, dst_hbm, ones_l_hbm, ones_r_hbm, zeros_hbm,
                   deg_hbm, srcv, dstv, ones_l, ones_r, acc, sem0, sem1):
        c = lax.axis_index("c")
        s = lax.axis_index("s")
        wid = s * NC + c
        r0 = s * rpt
        pltpu.sync_copy(zeros_hbm.at[pl.ds(r0, rpt)], acc.at[pl.ds(r0, rpt)])
        pltpu.sync_copy(ones_l_hbm, ones_l)
        pltpu.sync_copy(ones_r_hbm, ones_r)
        plsc.subcore_barrier()

        def slab(si, carry):
            base = wid * ch + si * SLAB
            pltpu.sync_copy(src_hbm.at[pl.ds(base, SLAB)], srcv)
            pltpu.sync_copy(dst_hbm.at[pl.ds(base, SLAB)], dstv)

            pltpu.async_copy(ones_l, acc.at[srcv.at[0]], sem0, add=True)
            pltpu.async_copy(ones_r, acc.at[dstv.at[0]], sem1, add=True)

            def step(j, c2):
                @pl.when(j < SLAB - 1)
                def _():
                    pltpu.async_copy(ones_l, acc.at[srcv.at[j + 1]], sem0, add=True)
                    pltpu.async_copy(ones_r, acc.at[dstv.at[j + 1]], sem1, add=True)
                pltpu.make_async_copy(ones_l, acc.at[srcv.at[j]], sem0).wait()
                pltpu.make_async_copy(ones_r, acc.at[dstv.at[j]], sem1).wait()
                return c2

            lax.fori_loop(0, SLAB, step, 0)
            return carry

        lax.fori_loop(0, ch // SLAB, slab, 0)
        plsc.subcore_barrier()
        pltpu.sync_copy(acc.at[pl.ds(r0, rpt)], deg_hbm.at[c, pl.ds(r0, rpt)])

    return deg_kernel


def _make_agg_kernel(pn, ch, d):
    rpt = pn // NS

    @functools.partial(
        pl.kernel,
        out_type=jax.ShapeDtypeStruct((NC, pn, d), jnp.float32),
        mesh=_mesh,
        scratch_types=[
            pltpu.VMEM((SLAB, CHUNK), jnp.int32),
            pltpu.VMEM((SLAB, CHUNK), jnp.int32),
            pltpu.VMEM((CHUNK, d), jnp.float32),
            pltpu.VMEM((CHUNK, d), jnp.float32),
            pltpu.VMEM_SHARED((pn, d), jnp.float32),
            pltpu.SemaphoreType.DMA,
            pltpu.SemaphoreType.DMA,
            pltpu.SemaphoreType.DMA,
            pltpu.SemaphoreType.DMA,
        ],
    )
    def agg_kernel(table_hbm, src_hbm, dst_hbm, zeros_hbm, out_hbm,
                   srcv, dstv, rows0, rows1, acc, semg0, semg1, sems0, sems1):
        c = lax.axis_index("c")
        s = lax.axis_index("s")
        wid = s * NC + c
        r0 = s * rpt
        pltpu.sync_copy(zeros_hbm.at[pl.ds(r0, rpt)], acc.at[pl.ds(r0, rpt)])
        plsc.subcore_barrier()

        def slab(si, carry):
            base = wid * ch + si * SLAB
            pltpu.sync_copy(src_hbm.at[pl.ds(base, SLAB)], srcv)
            pltpu.sync_copy(dst_hbm.at[pl.ds(base, SLAB)], dstv)
            pltpu.async_copy(table_hbm.at[srcv.at[0]], rows0, semg0)

            def step(jj, c2):
                j0 = 2 * jj
                j1 = j0 + 1
                pltpu.async_copy(table_hbm.at[srcv.at[j1]], rows1, semg1)
                pltpu.make_async_copy(table_hbm.at[srcv.at[j0]], rows0, semg0).wait()
                pltpu.sync_copy(rows0, acc.at[dstv.at[j0]], add=True)
                nxt = jnp.minimum(j0 + 2, SLAB - 1)
                pltpu.async_copy(table_hbm.at[srcv.at[nxt]], rows0, semg0)
                pltpu.make_async_copy(table_hbm.at[srcv.at[j1]], rows1, semg1).wait()
                pltpu.sync_copy(rows1, acc.at[dstv.at[j1]], add=True)
                return c2

            lax.fori_loop(0, SLAB // 2, step, 0)
            pltpu.make_async_copy(table_hbm.at[srcv.at[0]], rows0, semg0).wait()
            return carry

        lax.fori_loop(0, ch // SLAB, slab, 0)
        plsc.subcore_barrier()
        pltpu.sync_copy(acc.at[pl.ds(r0, rpt)], out_hbm.at[c, pl.ds(r0, rpt)])

    return agg_kernel


def _hprime_body(x_ref, deg_ref, w1_ref, out_ref):
    d = deg_ref[0, :, 0:1] + deg_ref[1, :, 0:1]
    norm = lax.rsqrt(jnp.maximum(d, 1.0))
    out_ref[...] = jnp.dot(x_ref[...] * norm, w1_ref[...],
                           preferred_element_type=jnp.float32)


def _mid_body(agg_ref, deg_ref, b1_ref, ws_ref, wn_ref, b2_ref, s_ref, p_ref):
    a = agg_ref[0] + agg_ref[1]
    din = deg_ref[0, :, 0:1] + deg_ref[1, :, 0:1]
    norm_in = lax.rsqrt(jnp.maximum(din, 1.0))
    h1 = jnp.maximum(a * norm_in + b1_ref[...], 0.0)
    s_ref[...] = jnp.dot(h1, ws_ref[...],
                         preferred_element_type=jnp.float32) + b2_ref[...]
    p_ref[...] = jnp.dot(h1, wn_ref[...], preferred_element_type=jnp.float32)


def _final_body(s_ref, nb_ref, deg_ref, out_ref):
    din = deg_ref[0, :, 0:1] + deg_ref[1, :, 0:1]
    inv = 1.0 / jnp.maximum(din, 1.0)
    out_ref[...] = s_ref[...] + (nb_ref[0] + nb_ref[1]) * inv


def kernel(x, edge_index, W1, b1, W_self, W_neigh, b2):
    n, d = x.shape
    e = edge_index.shape[1]
    pn = _round_up(n, 512)
    ch = _round_up(-(-e // (NW * CHUNK)), SLAB)
    pad_e = NW * ch * CHUNK

    src = edge_index[0].astype(jnp.int32)
    dst = edge_index[1].astype(jnp.int32)
    n_sent = pn - n
    sent = n + (jnp.arange(pad_e - e, dtype=jnp.int32) % n_sent)
    src2d = jnp.concatenate([src, sent]).reshape(NW * ch, CHUNK)
    dst2d = jnp.concatenate([dst, sent]).reshape(NW * ch, CHUNK)
    x_pad = jnp.concatenate([x, jnp.zeros((pn - n, d), x.dtype)])

    zeros_d = jnp.zeros((pn, d), jnp.float32)
    hd = d // 2
    ones_l = jnp.concatenate([jnp.ones((CHUNK, hd), jnp.float32),
                              jnp.zeros((CHUNK, hd), jnp.float32)], axis=1)
    ones_r = ones_l[:, ::-1]

    agg_kern = _make_agg_kernel(pn, ch, d)
    deg = _make_deg_kernel(pn, ch, d)(src2d, dst2d, ones_l, ones_r, zeros_d)
    deg_o_col = deg[:, :, 0:8]
    deg_i_col = deg[:, :, 64:72]

    br = pn // 4
    grid = pn // br
    row_spec = pl.BlockSpec((br, d), lambda i: (i, 0))
    two_spec = pl.BlockSpec((NC, br, d), lambda i: (0, i, 0))
    deg_spec = pl.BlockSpec((NC, br, 8), lambda i: (0, i, 0))
    w_spec = pl.BlockSpec((d, d), lambda i: (0, 0))
    b_spec = pl.BlockSpec((1, d), lambda i: (0, 0))
    fs = jax.ShapeDtypeStruct((pn, d), jnp.float32)

    hprime = pl.pallas_call(
        _hprime_body, grid=(grid,),
        in_specs=[row_spec, deg_spec, w_spec],
        out_specs=row_spec, out_shape=fs,
    )(x_pad, deg_o_col, W1)

    agg = agg_kern(hprime, src2d, dst2d, zeros_d)

    s_out, p_out = pl.pallas_call(
        _mid_body, grid=(grid,),
        in_specs=[two_spec, deg_spec, b_spec, w_spec, w_spec, b_spec],
        out_specs=(row_spec, row_spec), out_shape=(fs, fs),
    )(agg, deg_i_col, b1.reshape(1, d), W_self, W_neigh, b2.reshape(1, d))

    neigh = agg_kern(p_out, src2d, dst2d, zeros_d)

    h2 = pl.pallas_call(
        _final_body, grid=(grid,),
        in_specs=[row_spec, two_spec, deg_spec],
        out_specs=row_spec, out_shape=fs,
    )(s_out, neigh, deg_i_col)

    return h2[:n]


# speedup vs baseline: 1.0373x; 1.0334x over previous
import functools

import jax
import jax.numpy as jnp
from jax import lax
from jax.experimental import pallas as pl
from jax.experimental.pallas import tpu as pltpu
from jax.experimental.pallas import tpu_sc as plsc

NC = 2
NS = 16
NW = NC * NS
CHUNK = 128
SLAB = 16

_mesh = plsc.VectorSubcoreMesh(core_axis_name="c", subcore_axis_name="s")


def _round_up(a, b):
    return (a + b - 1) // b * b


def _make_deg_kernel(pn, ch, d):
    rpt = pn // NS

    @functools.partial(
        pl.kernel,
        out_type=jax.ShapeDtypeStruct((NC, pn, d), jnp.float32),
        mesh=_mesh,
        scratch_types=[
            pltpu.VMEM((SLAB, CHUNK), jnp.int32),
            pltpu.VMEM((SLAB, CHUNK), jnp.int32),
            pltpu.VMEM((CHUNK, d), jnp.float32),
            pltpu.VMEM((CHUNK, d), jnp.float32),
            pltpu.VMEM_SHARED((pn, d), jnp.float32),
            pltpu.SemaphoreType.DMA,
            pltpu.SemaphoreType.DMA,
        ],
    )
    def deg_kernel(src_hbm, dst_hbm, ones_l_hbm, ones_r_hbm, zeros_hbm,
                   deg_hbm, srcv, dstv, ones_l, ones_r, acc, sem0, sem1):
        c = lax.axis_index("c")
        s = lax.axis_index("s")
        wid = s * NC + c
        r0 = s * rpt
        pltpu.sync_copy(zeros_hbm.at[pl.ds(r0, rpt)], acc.at[pl.ds(r0, rpt)])
        pltpu.sync_copy(ones_l_hbm, ones_l)
        pltpu.sync_copy(ones_r_hbm, ones_r)
        plsc.subcore_barrier()

        def slab(si, carry):
            base = wid * ch + si * SLAB
            pltpu.sync_copy(src_hbm.at[pl.ds(base, SLAB)], srcv)
            pltpu.sync_copy(dst_hbm.at[pl.ds(base, SLAB)], dstv)

            pltpu.async_copy(ones_l, acc.at[srcv.at[0]], sem0, add=True)
            pltpu.async_copy(ones_r, acc.at[dstv.at[0]], sem1, add=True)

            def step(j, c2):
                @pl.when(j < SLAB - 1)
                def _():
                    pltpu.async_copy(ones_l, acc.at[srcv.at[j + 1]], sem0, add=True)
                    pltpu.async_copy(ones_r, acc.at[dstv.at[j + 1]], sem1, add=True)
                pltpu.make_async_copy(ones_l, acc.at[srcv.at[j]], sem0).wait()
                pltpu.make_async_copy(ones_r, acc.at[dstv.at[j]], sem1).wait()
                return c2

            lax.fori_loop(0, SLAB, step, 0)
            return carry

        lax.fori_loop(0, ch // SLAB, slab, 0)
        plsc.subcore_barrier()
        pltpu.sync_copy(acc.at[pl.ds(r0, rpt)], deg_hbm.at[c, pl.ds(r0, rpt)])

    return deg_kernel


def _make_agg_kernel(pn, ch, d):
    rpt = pn // NS

    @functools.partial(
        pl.kernel,
        out_type=jax.ShapeDtypeStruct((NC, pn, d), jnp.float32),
        mesh=_mesh,
        scratch_types=[
            pltpu.VMEM((SLAB, CHUNK), jnp.int32),
            pltpu.VMEM((SLAB, CHUNK), jnp.int32),
            pltpu.VMEM((CHUNK, d), jnp.float32),
            pltpu.VMEM((CHUNK, d), jnp.float32),
            pltpu.VMEM_SHARED((pn, d), jnp.float32),
            pltpu.SemaphoreType.DMA,
            pltpu.SemaphoreType.DMA,
            pltpu.SemaphoreType.DMA,
            pltpu.SemaphoreType.DMA,
        ],
    )
    def agg_kernel(table_hbm, src_hbm, dst_hbm, zeros_hbm, out_hbm,
                   srcv, dstv, rows0, rows1, acc, semg0, semg1, sems0, sems1):
        c = lax.axis_index("c")
        s = lax.axis_index("s")
        wid = s * NC + c
        r0 = s * rpt
        pltpu.sync_copy(zeros_hbm.at[pl.ds(r0, rpt)], acc.at[pl.ds(r0, rpt)])
        plsc.subcore_barrier()

        def slab(si, carry):
            base = wid * ch + si * SLAB
            pltpu.sync_copy(src_hbm.at[pl.ds(base, SLAB)], srcv)
            pltpu.sync_copy(dst_hbm.at[pl.ds(base, SLAB)], dstv)
            pltpu.async_copy(table_hbm.at[srcv.at[0]], rows0, semg0)

            def step(jj, c2):
                j0 = 2 * jj
                j1 = j0 + 1
                pltpu.async_copy(table_hbm.at[srcv.at[j1]], rows1, semg1)
                pltpu.make_async_copy(table_hbm.at[srcv.at[j0]], rows0, semg0).wait()
                pltpu.sync_copy(rows0, acc.at[dstv.at[j0]], add=True)
                nxt = jnp.minimum(j0 + 2, SLAB - 1)
                pltpu.async_copy(table_hbm.at[srcv.at[nxt]], rows0, semg0)
                pltpu.make_async_copy(table_hbm.at[srcv.at[j1]], rows1, semg1).wait()
                pltpu.sync_copy(rows1, acc.at[dstv.at[j1]], add=True)
                return c2

            lax.fori_loop(0, SLAB // 2, step, 0)
            pltpu.make_async_copy(table_hbm.at[srcv.at[0]], rows0, semg0).wait()
            return carry

        lax.fori_loop(0, ch // SLAB, slab, 0)
        plsc.subcore_barrier()
        pltpu.sync_copy(acc.at[pl.ds(r0, rpt)], out_hbm.at[c, pl.ds(r0, rpt)])

    return agg_kernel


def _hprime_body(x_ref, deg_ref, w1_ref, out_ref):
    d = deg_ref[0, :, 0:1] + deg_ref[1, :, 0:1]
    norm = lax.rsqrt(jnp.maximum(d, 1.0))
    out_ref[...] = jnp.dot(x_ref[...] * norm, w1_ref[...],
                           preferred_element_type=jnp.float32)


def _mid_body(agg_ref, deg_ref, b1_ref, ws_ref, wn_ref, b2_ref, s_ref, p_ref):
    a = agg_ref[0] + agg_ref[1]
    din = deg_ref[0, :, 64:65] + deg_ref[1, :, 64:65]
    norm_in = lax.rsqrt(jnp.maximum(din, 1.0))
    h1 = jnp.maximum(a * norm_in + b1_ref[...], 0.0)
    s_ref[...] = jnp.dot(h1, ws_ref[...],
                         preferred_element_type=jnp.float32) + b2_ref[...]
    p_ref[...] = jnp.dot(h1, wn_ref[...], preferred_element_type=jnp.float32)


def _final_body(s_ref, nb_ref, deg_ref, out_ref):
    din = deg_ref[0, :, 64:65] + deg_ref[1, :, 64:65]
    inv = 1.0 / jnp.maximum(din, 1.0)
    out_ref[...] = s_ref[...] + (nb_ref[0] + nb_ref[1]) * inv


def kernel(x, edge_index, W1, b1, W_self, W_neigh, b2):
    n, d = x.shape
    e = edge_index.shape[1]
    pn = _round_up(n, 512)
    ch = _round_up(-(-e // (NW * CHUNK)), SLAB)
    pad_e = NW * ch * CHUNK

    src = edge_index[0].astype(jnp.int32)
    dst = edge_index[1].astype(jnp.int32)
    n_sent = pn - n
    sent = n + (jnp.arange(pad_e - e, dtype=jnp.int32) % n_sent)
    src2d = jnp.concatenate([src, sent]).reshape(NW * ch, CHUNK)
    dst2d = jnp.concatenate([dst, sent]).reshape(NW * ch, CHUNK)
    x_pad = jnp.concatenate([x, jnp.zeros((pn - n, d), x.dtype)])

    zeros_d = jnp.zeros((pn, d), jnp.float32)
    hd = d // 2
    ones_l = jnp.concatenate([jnp.ones((CHUNK, hd), jnp.float32),
                              jnp.zeros((CHUNK, hd), jnp.float32)], axis=1)
    ones_r = ones_l[:, ::-1]

    agg_kern = _make_agg_kernel(pn, ch, d)
    deg = _make_deg_kernel(pn, ch, d)(src2d, dst2d, ones_l, ones_r, zeros_d)

    br = pn // 4
    grid = pn // br
    row_spec = pl.BlockSpec((br, d), lambda i: (i, 0))
    two_spec = pl.BlockSpec((NC, br, d), lambda i: (0, i, 0))
    deg_spec = two_spec
    w_spec = pl.BlockSpec((d, d), lambda i: (0, 0))
    b_spec = pl.BlockSpec((1, d), lambda i: (0, 0))
    fs = jax.ShapeDtypeStruct((pn, d), jnp.float32)

    hprime = pl.pallas_call(
        _hprime_body, grid=(grid,),
        in_specs=[row_spec, deg_spec, w_spec],
        out_specs=row_spec, out_shape=fs,
    )(x_pad, deg, W1)

    agg = agg_kern(hprime, src2d, dst2d, zeros_d)

    s_out, p_out = pl.pallas_call(
        _mid_body, grid=(grid,),
        in_specs=[two_spec, deg_spec, b_spec, w_spec, w_spec, b_spec],
        out_specs=(row_spec, row_spec), out_shape=(fs, fs),
    )(agg, deg, b1.reshape(1, d), W_self, W_neigh, b2.reshape(1, d))

    neigh = agg_kern(p_out, src2d, dst2d, zeros_d)

    h2 = pl.pallas_call(
        _final_body, grid=(grid,),
        in_specs=[row_spec, two_spec, deg_spec],
        out_specs=row_spec, out_shape=fs,
    )(s_out, neigh, deg)

    return h2[:n]


# speedup vs baseline: 1.0974x; 1.0579x over previous
import functools

import jax
import jax.numpy as jnp
from jax import lax
from jax.experimental import pallas as pl
from jax.experimental.pallas import tpu as pltpu
from jax.experimental.pallas import tpu_sc as plsc

NC = 2
NS = 16
NW = NC * NS
CHUNK = 128
SLAB = 40

_mesh = plsc.VectorSubcoreMesh(core_axis_name="c", subcore_axis_name="s")


def _round_up(a, b):
    return (a + b - 1) // b * b


def _make_deg_kernel(pn, ch, d):
    rpt = pn // NS

    @functools.partial(
        pl.kernel,
        out_type=jax.ShapeDtypeStruct((NC, pn, d), jnp.float32),
        mesh=_mesh,
        scratch_types=[
            pltpu.VMEM((SLAB, CHUNK), jnp.int32),
            pltpu.VMEM((SLAB, CHUNK), jnp.int32),
            pltpu.VMEM((CHUNK, d), jnp.float32),
            pltpu.VMEM((CHUNK, d), jnp.float32),
            pltpu.VMEM_SHARED((pn, d), jnp.float32),
            pltpu.SemaphoreType.DMA,
            pltpu.SemaphoreType.DMA,
        ],
    )
    def deg_kernel(src_hbm, dst_hbm, ones_l_hbm, ones_r_hbm, zeros_hbm,
                   deg_hbm, srcv, dstv, ones_l, ones_r, acc, sem0, sem1):
        c = lax.axis_index("c")
        s = lax.axis_index("s")
        wid = s * NC + c
        r0 = s * rpt
        pltpu.sync_copy(zeros_hbm.at[pl.ds(r0, rpt)], acc.at[pl.ds(r0, rpt)])
        pltpu.sync_copy(ones_l_hbm, ones_l)
        pltpu.sync_copy(ones_r_hbm, ones_r)
        plsc.subcore_barrier()

        def slab(si, carry):
            base = wid * ch + si * SLAB
            pltpu.sync_copy(src_hbm.at[pl.ds(base, SLAB)], srcv)
            pltpu.sync_copy(dst_hbm.at[pl.ds(base, SLAB)], dstv)

            pltpu.async_copy(ones_l, acc.at[srcv.at[0]], sem0, add=True)
            pltpu.async_copy(ones_r, acc.at[dstv.at[0]], sem1, add=True)

            def step(j, c2):
                @pl.when(j < SLAB - 1)
                def _():
                    pltpu.async_copy(ones_l, acc.at[srcv.at[j + 1]], sem0, add=True)
                    pltpu.async_copy(ones_r, acc.at[dstv.at[j + 1]], sem1, add=True)
                pltpu.make_async_copy(ones_l, acc.at[srcv.at[j]], sem0).wait()
                pltpu.make_async_copy(ones_r, acc.at[dstv.at[j]], sem1).wait()
                return c2

            lax.fori_loop(0, SLAB, step, 0)
            return carry

        lax.fori_loop(0, ch // SLAB, slab, 0)
        plsc.subcore_barrier()
        pltpu.sync_copy(acc.at[pl.ds(r0, rpt)], deg_hbm.at[c, pl.ds(r0, rpt)])

    return deg_kernel


def _make_agg_kernel(pn, ch, d):
    rpt = pn // NS

    @functools.partial(
        pl.kernel,
        out_type=jax.ShapeDtypeStruct((NC, pn, d), jnp.float32),
        mesh=_mesh,
        scratch_types=[
            pltpu.VMEM((SLAB, CHUNK), jnp.int32),
            pltpu.VMEM((SLAB, CHUNK), jnp.int32),
            pltpu.VMEM((CHUNK, d), jnp.float32),
            pltpu.VMEM((CHUNK, d), jnp.float32),
            pltpu.VMEM_SHARED((pn, d), jnp.float32),
            pltpu.SemaphoreType.DMA,
            pltpu.SemaphoreType.DMA,
            pltpu.SemaphoreType.DMA,
            pltpu.SemaphoreType.DMA,
        ],
    )
    def agg_kernel(table_hbm, src_hbm, dst_hbm, zeros_hbm, out_hbm,
                   srcv, dstv, rows0, rows1, acc, semg0, semg1, sems0, sems1):
        c = lax.axis_index("c")
        s = lax.axis_index("s")
        wid = s * NC + c
        r0 = s * rpt
        pltpu.sync_copy(zeros_hbm.at[pl.ds(r0, rpt)], acc.at[pl.ds(r0, rpt)])
        plsc.subcore_barrier()

        def slab(si, carry):
            base = wid * ch + si * SLAB
            pltpu.sync_copy(src_hbm.at[pl.ds(base, SLAB)], srcv)
            pltpu.sync_copy(dst_hbm.at[pl.ds(base, SLAB)], dstv)
            pltpu.async_copy(table_hbm.at[srcv.at[0]], rows0, semg0)

            def step(jj, c2):
                j0 = 2 * jj
                j1 = j0 + 1
                pltpu.async_copy(table_hbm.at[srcv.at[j1]], rows1, semg1)
                pltpu.make_async_copy(table_hbm.at[srcv.at[j0]], rows0, semg0).wait()
                pltpu.sync_copy(rows0, acc.at[dstv.at[j0]], add=True)
                nxt = jnp.minimum(j0 + 2, SLAB - 1)
                pltpu.async_copy(table_hbm.at[srcv.at[nxt]], rows0, semg0)
                pltpu.make_async_copy(table_hbm.at[srcv.at[j1]], rows1, semg1).wait()
                pltpu.sync_copy(rows1, acc.at[dstv.at[j1]], add=True)
                return c2

            lax.fori_loop(0, SLAB // 2, step, 0)
            pltpu.make_async_copy(table_hbm.at[srcv.at[0]], rows0, semg0).wait()
            return carry

        lax.fori_loop(0, ch // SLAB, slab, 0)
        plsc.subcore_barrier()
        pltpu.sync_copy(acc.at[pl.ds(r0, rpt)], out_hbm.at[c, pl.ds(r0, rpt)])

    return agg_kernel


def _hprime_body(x_ref, deg_ref, w1_ref, out_ref):
    d = deg_ref[0, :, 0:1] + deg_ref[1, :, 0:1]
    norm = lax.rsqrt(jnp.maximum(d, 1.0))
    out_ref[...] = jnp.dot(x_ref[...] * norm, w1_ref[...],
                           preferred_element_type=jnp.float32)


def _mid_body(agg_ref, deg_ref, b1_ref, ws_ref, wn_ref, b2_ref, s_ref, p_ref):
    a = agg_ref[0] + agg_ref[1]
    din = deg_ref[0, :, 64:65] + deg_ref[1, :, 64:65]
    norm_in = lax.rsqrt(jnp.maximum(din, 1.0))
    h1 = jnp.maximum(a * norm_in + b1_ref[...], 0.0)
    s_ref[...] = jnp.dot(h1, ws_ref[...],
                         preferred_element_type=jnp.float32) + b2_ref[...]
    p_ref[...] = jnp.dot(h1, wn_ref[...], preferred_element_type=jnp.float32)


def _final_body(s_ref, nb_ref, deg_ref, out_ref):
    din = deg_ref[0, :, 64:65] + deg_ref[1, :, 64:65]
    inv = 1.0 / jnp.maximum(din, 1.0)
    out_ref[...] = s_ref[...] + (nb_ref[0] + nb_ref[1]) * inv


def kernel(x, edge_index, W1, b1, W_self, W_neigh, b2):
    n, d = x.shape
    e = edge_index.shape[1]
    pn = _round_up(n, 512)
    ch = _round_up(-(-e // (NW * CHUNK)), SLAB)
    pad_e = NW * ch * CHUNK

    src = edge_index[0].astype(jnp.int32)
    dst = edge_index[1].astype(jnp.int32)
    n_sent = pn - n
    sent = n + (jnp.arange(pad_e - e, dtype=jnp.int32) % n_sent)
    src2d = jnp.concatenate([src, sent]).reshape(NW * ch, CHUNK)
    dst2d = jnp.concatenate([dst, sent]).reshape(NW * ch, CHUNK)
    x_pad = jnp.concatenate([x, jnp.zeros((pn - n, d), x.dtype)])

    zeros_d = jnp.zeros((pn, d), jnp.float32)
    hd = d // 2
    ones_l = jnp.concatenate([jnp.ones((CHUNK, hd), jnp.float32),
                              jnp.zeros((CHUNK, hd), jnp.float32)], axis=1)
    ones_r = ones_l[:, ::-1]

    agg_kern = _make_agg_kernel(pn, ch, d)
    deg = _make_deg_kernel(pn, ch, d)(src2d, dst2d, ones_l, ones_r, zeros_d)

    br = pn // 4
    grid = pn // br
    row_spec = pl.BlockSpec((br, d), lambda i: (i, 0))
    two_spec = pl.BlockSpec((NC, br, d), lambda i: (0, i, 0))
    deg_spec = two_spec
    w_spec = pl.BlockSpec((d, d), lambda i: (0, 0))
    b_spec = pl.BlockSpec((1, d), lambda i: (0, 0))
    fs = jax.ShapeDtypeStruct((pn, d), jnp.float32)

    hprime = pl.pallas_call(
        _hprime_body, grid=(grid,),
        in_specs=[row_spec, deg_spec, w_spec],
        out_specs=row_spec, out_shape=fs,
    )(x_pad, deg, W1)

    agg = agg_kern(hprime, src2d, dst2d, zeros_d)

    s_out, p_out = pl.pallas_call(
        _mid_body, grid=(grid,),
        in_specs=[two_spec, deg_spec, b_spec, w_spec, w_spec, b_spec],
        out_specs=(row_spec, row_spec), out_shape=(fs, fs),
    )(agg, deg, b1.reshape(1, d), W_self, W_neigh, b2.reshape(1, d))

    neigh = agg_kern(p_out, src2d, dst2d, zeros_d)

    h2 = pl.pallas_call(
        _final_body, grid=(grid,),
        in_specs=[row_spec, two_spec, deg_spec],
        out_specs=row_spec, out_shape=fs,
    )(s_out, neigh, deg)

    return h2[:n]
